# baseline (speedup 1.0000x reference)
import functools

import jax
import jax.numpy as jnp
from jax import lax
from jax.experimental import pallas as pl
from jax.experimental.pallas import tpu as pltpu
from jax.experimental.pallas import tpu_sc as plsc

N = 10000
D = 128
C = 40
E = 320000
NW = 32
BATCH = 128
CH0 = 104
CH1 = 53
CHMAX = max(CH0, CH1)
E_PAD = 16 * (CH0 + CH1) * BATCH
NP = N + 112
STRIPE = NP // 16

_MESH = plsc.VectorSubcoreMesh(core_axis_name="c", subcore_axis_name="s")


@functools.partial(
    pl.kernel,
    out_type=jax.ShapeDtypeStruct((2, NP, D), jnp.float32),
    mesh=_MESH,
    scratch_types=[
        pltpu.VMEM((CHMAX, BATCH), jnp.int32),
        pltpu.VMEM((BATCH, D), jnp.float32),
        pltpu.VMEM_SHARED((NP, D), jnp.float32),
        pltpu.SemaphoreType.DMA,
    ],
)
def _deg_kernel(dst_hbm, ones_hbm, z_hbm, out_hbm, dst_v, ones_v, acc, sem):
    c = lax.axis_index("c")
    s = lax.axis_index("s")
    nc = jnp.where(c == 0, CH0, CH1)
    pltpu.sync_copy(z_hbm, acc.at[pl.ds(s * STRIPE, STRIPE)])
    pltpu.sync_copy(ones_hbm, ones_v)
    pltpu.sync_copy(dst_hbm.at[s, c], dst_v)
    plsc.subcore_barrier()

    def body(j, carry):
        pltpu.async_copy(ones_v, acc.at[dst_v.at[j]], sem, add=True)

        @pl.when(j >= 8)
        def _():
            pltpu.make_async_copy(ones_v, acc.at[dst_v.at[j - 8]], sem).wait()

        return carry

    lax.fori_loop(0, nc, body, 0)

    def drain(j, carry):
        pltpu.make_async_copy(ones_v, acc.at[dst_v.at[j]], sem).wait()
        return carry

    lax.fori_loop(nc - 8, nc, drain, 0)
    plsc.subcore_barrier()
    pltpu.sync_copy(acc.at[pl.ds(s * STRIPE, STRIPE)],
                    out_hbm.at[c, pl.ds(s * STRIPE, STRIPE)])


@functools.partial(
    pl.kernel,
    out_type=jax.ShapeDtypeStruct((2, NP, D), jnp.float32),
    mesh=_MESH,
    scratch_types=[
        pltpu.VMEM((CHMAX, BATCH), jnp.int32),
        pltpu.VMEM((CHMAX, BATCH), jnp.int32),
        pltpu.VMEM((BATCH, D), jnp.float32),
        pltpu.VMEM_SHARED((NP, D), jnp.float32),
        pltpu.SemaphoreType.DMA,
    ],
)
def _agg_kernel(q_hbm, src_hbm, dst_hbm, z_hbm, out_hbm,
                src_v, dst_v, rows_v, acc, sem):
    c = lax.axis_index("c")
    s = lax.axis_index("s")
    nc = jnp.where(c == 0, CH0, CH1)
    pltpu.sync_copy(z_hbm, acc.at[pl.ds(s * STRIPE, STRIPE)])
    pltpu.sync_copy(src_hbm.at[s, c], src_v)
    pltpu.sync_copy(dst_hbm.at[s, c], dst_v)
    plsc.subcore_barrier()

    def body(j, carry):
        pltpu.async_copy(q_hbm.at[src_v.at[j]], rows_v, sem).wait()
        pltpu.sync_copy(rows_v, acc.at[dst_v.at[j]], add=True)
        return carry

    lax.fori_loop(0, nc, body, 0)
    plsc.subcore_barrier()
    pltpu.sync_copy(acc.at[pl.ds(s * STRIPE, STRIPE)],
                    out_hbm.at[c, pl.ds(s * STRIPE, STRIPE)])


def _dis(degp_ref):
    deg = degp_ref[0, :N, 0:1] + degp_ref[1, :N, 0:1] + 1.0
    return lax.rsqrt(deg)


def _pad16(q):
    return jnp.concatenate([q, jnp.zeros((NP - N, D), jnp.float32)], axis=0)


def _pre_body(x_ref, w_ref, degp_ref, q_ref):
    dis = _dis(degp_ref)
    q = jnp.dot(x_ref[...], w_ref[...],
                preferred_element_type=jnp.float32) * dis
    q_ref[...] = _pad16(q)


def _bn_act(part_ref, qp_ref, degp_ref, b_ref, g_ref, be_ref):
    dis = _dis(degp_ref)
    agg = part_ref[0, :N, :] + part_ref[1, :N, :] + qp_ref[:N, :]
    out = agg * dis + b_ref[...]
    m = jnp.mean(out, axis=0, keepdims=True)
    v = jnp.mean((out - m) ** 2, axis=0, keepdims=True)
    h = (out - m) * lax.rsqrt(v + 1e-5) * g_ref[...] + be_ref[...]
    return jnp.where(h >= 0, h, 0.01 * h), dis


def _mid_body(part_ref, qp_ref, degp_ref, b_ref, g_ref, be_ref, w_ref, q_ref):
    h, dis = _bn_act(part_ref, qp_ref, degp_ref, b_ref, g_ref, be_ref)
    q = jnp.dot(h, w_ref[...], preferred_element_type=jnp.float32) * dis
    q_ref[...] = _pad16(q)


def _fin_body(part_ref, qp_ref, degp_ref, b_ref, g_ref, be_ref,
              wfc_ref, bfc_ref, out_ref):
    h, _ = _bn_act(part_ref, qp_ref, degp_ref, b_ref, g_ref, be_ref)
    logits = jnp.dot(h, wfc_ref[...],
                     preferred_element_type=jnp.float32) + bfc_ref[...]
    mx = jnp.max(logits, axis=1, keepdims=True)
    z = logits - mx
    out_ref[...] = z - jnp.log(jnp.sum(jnp.exp(z), axis=1, keepdims=True))


_pre = pl.pallas_call(
    _pre_body, out_shape=jax.ShapeDtypeStruct((NP, D), jnp.float32))
_mid = pl.pallas_call(
    _mid_body, out_shape=jax.ShapeDtypeStruct((NP, D), jnp.float32))
_fin = pl.pallas_call(
    _fin_body, out_shape=jax.ShapeDtypeStruct((N, C), jnp.float32))


def kernel(x, edge_index, W1, b1, g1, be1, W2, b2, g2, be2,
           W3, b3, g3, be3, Wfc, bfc):
    n0 = 16 * CH0 * BATCH
    n1 = 16 * CH1 * BATCH

    def _split(v, fill):
        vp = jnp.concatenate([v, jnp.full((E_PAD - E,), fill, jnp.int32)])
        c0 = vp[:n0].reshape(16, 1, CH0, BATCH)
        c1 = vp[n0:].reshape(16, 1, CH1, BATCH)
        pad0 = ((0, 0), (0, 0), (0, CHMAX - CH0), (0, 0))
        pad1 = ((0, 0), (0, 0), (0, CHMAX - CH1), (0, 0))
        return jnp.concatenate([jnp.pad(c0, pad0, constant_values=fill),
                                jnp.pad(c1, pad1, constant_values=fill)],
                               axis=1)

    src = _split(edge_index[0], 0)
    dst = _split(edge_index[1], N)
    zD = x[:STRIPE] * 0.0
    onesB = x[:BATCH] * 0.0 + 1.0

    degp = _deg_kernel(dst, onesB, zD)
    q1 = _pre(x, W1, degp)
    p1 = _agg_kernel(q1, src, dst, zD)
    q2 = _mid(p1, q1, degp, b1, g1, be1, W2)
    p2 = _agg_kernel(q2, src, dst, zD)
    q3 = _mid(p2, q2, degp, b2, g2, be2, W3)
    p3 = _agg_kernel(q3, src, dst, zD)
    return _fin(p3, q3, degp, b3, g3, be3, Wfc, bfc)


# speedup vs baseline: 1.0570x; 1.0570x over previous
import functools

import jax
import jax.numpy as jnp
from jax import lax
from jax.experimental import pallas as pl
from jax.experimental.pallas import tpu as pltpu
from jax.experimental.pallas import tpu_sc as plsc

N = 10000
D = 128
C = 40
E = 320000
NW = 32
BATCH = 128
CH0 = 96
CH1 = 61
CHMAX = max(CH0, CH1)
E_PAD = 16 * (CH0 + CH1) * BATCH
NP = N + 112
STRIPE = NP // 16

_MESH = plsc.VectorSubcoreMesh(core_axis_name="c", subcore_axis_name="s")


@functools.partial(
    pl.kernel,
    out_type=jax.ShapeDtypeStruct((2, NP, D), jnp.float32),
    mesh=_MESH,
    scratch_types=[
        pltpu.VMEM((CHMAX, BATCH), jnp.int32),
        pltpu.VMEM((BATCH, D), jnp.float32),
        pltpu.VMEM_SHARED((NP, D), jnp.float32),
        pltpu.SemaphoreType.DMA,
    ],
)
def _deg_kernel(dst_hbm, ones_hbm, z_hbm, out_hbm, dst_v, ones_v, acc, sem):
    c = lax.axis_index("c")
    s = lax.axis_index("s")
    nc = jnp.where(c == 0, CH0, CH1)
    pltpu.sync_copy(z_hbm, acc.at[pl.ds(s * STRIPE, STRIPE)])
    pltpu.sync_copy(ones_hbm, ones_v)
    pltpu.sync_copy(dst_hbm.at[s, c], dst_v)
    plsc.subcore_barrier()

    def body(j, carry):
        pltpu.async_copy(ones_v, acc.at[dst_v.at[j]], sem, add=True)

        @pl.when(j >= 8)
        def _():
            pltpu.make_async_copy(ones_v, acc.at[dst_v.at[j - 8]], sem).wait()

        return carry

    lax.fori_loop(0, nc, body, 0)

    def drain(j, carry):
        pltpu.make_async_copy(ones_v, acc.at[dst_v.at[j]], sem).wait()
        return carry

    lax.fori_loop(nc - 8, nc, drain, 0)
    plsc.subcore_barrier()
    pltpu.sync_copy(acc.at[pl.ds(s * STRIPE, STRIPE)],
                    out_hbm.at[c, pl.ds(s * STRIPE, STRIPE)])


@functools.partial(
    pl.kernel,
    out_type=jax.ShapeDtypeStruct((2, NP, D), jnp.float32),
    mesh=_MESH,
    scratch_types=[
        pltpu.VMEM((CHMAX, BATCH), jnp.int32),
        pltpu.VMEM((CHMAX, BATCH), jnp.int32),
        pltpu.VMEM((BATCH, D), jnp.float32),
        pltpu.VMEM_SHARED((NP, D), jnp.float32),
        pltpu.SemaphoreType.DMA,
    ],
)
def _agg_kernel(q_hbm, src_hbm, dst_hbm, z_hbm, out_hbm,
                src_v, dst_v, rows_v, acc, sem):
    c = lax.axis_index("c")
    s = lax.axis_index("s")
    nc = jnp.where(c == 0, CH0, CH1)
    pltpu.sync_copy(z_hbm, acc.at[pl.ds(s * STRIPE, STRIPE)])
    pltpu.sync_copy(src_hbm.at[s, c], src_v)
    pltpu.sync_copy(dst_hbm.at[s, c], dst_v)
    plsc.subcore_barrier()

    def body(j, carry):
        pltpu.async_copy(q_hbm.at[src_v.at[j]], rows_v, sem).wait()
        pltpu.sync_copy(rows_v, acc.at[dst_v.at[j]], add=True)
        return carry

    lax.fori_loop(0, nc, body, 0)
    plsc.subcore_barrier()
    pltpu.sync_copy(acc.at[pl.ds(s * STRIPE, STRIPE)],
                    out_hbm.at[c, pl.ds(s * STRIPE, STRIPE)])


def _dis(degp_ref):
    deg = degp_ref[0, :N, 0:1] + degp_ref[1, :N, 0:1] + 1.0
    return lax.rsqrt(deg)


def _pad16(q):
    return jnp.concatenate([q, jnp.zeros((NP - N, D), jnp.float32)], axis=0)


def _pre_body(x_ref, w_ref, degp_ref, q_ref):
    dis = _dis(degp_ref)
    q = jnp.dot(x_ref[...], w_ref[...],
                preferred_element_type=jnp.float32) * dis
    q_ref[...] = _pad16(q)


def _bn_act(part_ref, qp_ref, degp_ref, b_ref, g_ref, be_ref):
    dis = _dis(degp_ref)
    agg = part_ref[0, :N, :] + part_ref[1, :N, :] + qp_ref[:N, :]
    out = agg * dis + b_ref[...]
    m = jnp.mean(out, axis=0, keepdims=True)
    v = jnp.mean((out - m) ** 2, axis=0, keepdims=True)
    h = (out - m) * lax.rsqrt(v + 1e-5) * g_ref[...] + be_ref[...]
    return jnp.where(h >= 0, h, 0.01 * h), dis


def _mid_body(part_ref, qp_ref, degp_ref, b_ref, g_ref, be_ref, w_ref, q_ref):
    h, dis = _bn_act(part_ref, qp_ref, degp_ref, b_ref, g_ref, be_ref)
    q = jnp.dot(h, w_ref[...], preferred_element_type=jnp.float32) * dis
    q_ref[...] = _pad16(q)


def _fin_body(part_ref, qp_ref, degp_ref, b_ref, g_ref, be_ref,
              wfc_ref, bfc_ref, out_ref):
    h, _ = _bn_act(part_ref, qp_ref, degp_ref, b_ref, g_ref, be_ref)
    logits = jnp.dot(h, wfc_ref[...],
                     preferred_element_type=jnp.float32) + bfc_ref[...]
    mx = jnp.max(logits, axis=1, keepdims=True)
    z = logits - mx
    out_ref[...] = z - jnp.log(jnp.sum(jnp.exp(z), axis=1, keepdims=True))


_pre = pl.pallas_call(
    _pre_body, out_shape=jax.ShapeDtypeStruct((NP, D), jnp.float32))
_mid = pl.pallas_call(
    _mid_body, out_shape=jax.ShapeDtypeStruct((NP, D), jnp.float32))
_fin = pl.pallas_call(
    _fin_body, out_shape=jax.ShapeDtypeStruct((N, C), jnp.float32))


def kernel(x, edge_index, W1, b1, g1, be1, W2, b2, g2, be2,
           W3, b3, g3, be3, Wfc, bfc):
    n0 = 16 * CH0 * BATCH
    n1 = 16 * CH1 * BATCH

    def _split(v, fill):
        vp = jnp.concatenate([v, jnp.full((E_PAD - E,), fill, jnp.int32)])
        c0 = vp[:n0].reshape(16, 1, CH0, BATCH)
        c1 = vp[n0:].reshape(16, 1, CH1, BATCH)
        pad0 = ((0, 0), (0, 0), (0, CHMAX - CH0), (0, 0))
        pad1 = ((0, 0), (0, 0), (0, CHMAX - CH1), (0, 0))
        return jnp.concatenate([jnp.pad(c0, pad0, constant_values=fill),
                                jnp.pad(c1, pad1, constant_values=fill)],
                               axis=1)

    src = _split(edge_index[0], 0)
    dst = _split(edge_index[1], N)
    zD = x[:STRIPE] * 0.0
    onesB = x[:BATCH] * 0.0 + 1.0

    degp = _deg_kernel(dst, onesB, zD)
    q1 = _pre(x, W1, degp)
    p1 = _agg_kernel(q1, src, dst, zD)
    q2 = _mid(p1, q1, degp, b1, g1, be1, W2)
    p2 = _agg_kernel(q2, src, dst, zD)
    q3 = _mid(p2, q2, degp, b2, g2, be2, W3)
    p3 = _agg_kernel(q3, src, dst, zD)
    return _fin(p3, q3, degp, b3, g3, be3, Wfc, bfc)
